# Initial kernel scaffold; baseline (speedup 1.0000x reference)
import jax
import jax.numpy as jnp
from jax.experimental import pallas as pl


def kernel(x, mask, F_emb, W1, b1, g1, bt1, W2, b2, g2, bt2, We1, be1, We2, be2):
    raise NotImplementedError("write your pallas kernel here")


# speedup vs baseline: 8.5502x; 8.5502x over previous
import jax
import jax.numpy as jnp
from jax.experimental import pallas as pl
from jax.experimental.pallas import tpu as pltpu

B, J, D, HH, EH, Z = 1024, 2048, 32, 128, 128, 64
TB, TJ = 8, 512
NB, NJ = B // TB, J // TJ


def _prep_kernel(F_ref, W10_ref, W1r_ref, G_ref):
    F = F_ref[...]
    nrm = jnp.sqrt(jnp.sum(F * F, axis=1, keepdims=True))
    Fn = F / jnp.maximum(nrm, 1e-8)
    G_ref[...] = W10_ref[...] + jnp.dot(
        Fn, W1r_ref[...], preferred_element_type=jnp.float32
    )


def _ln_rows(v, eps=1e-5):
    m = jnp.mean(v, axis=1, keepdims=True)
    c = v - m
    var = jnp.mean(c * c, axis=1, keepdims=True)
    return c * jax.lax.rsqrt(var + eps)


def _main_kernel(x_ref, m_ref, G_ref, W2_ref, b1_ref, g1_ref, bt1_ref,
                 b2_ref, g2_ref, bt2_ref, We1_ref, be1_ref, We2_ref, be2_ref,
                 mu_ref, lv_ref, acc, cnt):
    ij = pl.program_id(1)

    @pl.when(ij == 0)
    def _():
        acc[...] = jnp.zeros_like(acc)
        cnt[...] = jnp.zeros_like(cnt)

    xm = x_ref[...]
    mk = m_ref[...]
    g = G_ref[...]
    v = xm[:, :, None] * g[None, :, :]
    v = v.reshape(TB * TJ, HH) + b1_ref[...]
    h1 = jnp.maximum(_ln_rows(v) * g1_ref[...] + bt1_ref[...], 0.0)
    h2 = jnp.dot(h1, W2_ref[...], preferred_element_type=jnp.float32)
    h2 = h2 + b2_ref[...]
    h2 = jnp.maximum(_ln_rows(h2) * g2_ref[...] + bt2_ref[...], 0.0)
    h2 = h2.reshape(TB, TJ, D) * mk[:, :, None]
    acc[...] += jnp.sum(h2, axis=1)
    cnt[...] += jnp.sum(mk, axis=1, keepdims=True)

    @pl.when(ij == NJ - 1)
    def _():
        pooled = acc[...] / jnp.maximum(cnt[...], 1.0)
        e = jnp.dot(pooled, We1_ref[...], preferred_element_type=jnp.float32)
        e = jnp.maximum(_ln_rows(e + be1_ref[...]), 0.0)
        e = jnp.dot(e, We2_ref[...], preferred_element_type=jnp.float32)
        e = jnp.maximum(_ln_rows(e + be2_ref[...]), 0.0)
        mu_ref[...] = e[:, :Z]
        lv_ref[...] = e[:, Z:]


@jax.jit
def kernel(x, mask, F_emb, W1, b1, g1, bt1, W2, b2, g2, bt2, We1, be1, We2, be2):
    G = pl.pallas_call(
        _prep_kernel,
        out_shape=jax.ShapeDtypeStruct((J, HH), jnp.float32),
    )(F_emb, W1[0:1, :], W1[1:, :])

    mkf = mask.astype(jnp.float32)
    row = lambda a: a.reshape(1, -1)

    def const(shape):
        return pl.BlockSpec(shape, lambda ib, ij: (0, 0))

    mu, lv = pl.pallas_call(
        _main_kernel,
        grid=(NB, NJ),
        in_specs=[
            pl.BlockSpec((TB, TJ), lambda ib, ij: (ib, ij)),
            pl.BlockSpec((TB, TJ), lambda ib, ij: (ib, ij)),
            pl.BlockSpec((TJ, HH), lambda ib, ij: (ij, 0)),
            const((HH, D)),
            const((1, HH)), const((1, HH)), const((1, HH)),
            const((1, D)), const((1, D)), const((1, D)),
            const((D, EH)), const((1, EH)),
            const((EH, 2 * Z)), const((1, 2 * Z)),
        ],
        out_specs=[
            pl.BlockSpec((TB, Z), lambda ib, ij: (ib, 0)),
            pl.BlockSpec((TB, Z), lambda ib, ij: (ib, 0)),
        ],
        out_shape=[
            jax.ShapeDtypeStruct((B, Z), jnp.float32),
            jax.ShapeDtypeStruct((B, Z), jnp.float32),
        ],
        scratch_shapes=[
            pltpu.VMEM((TB, D), jnp.float32),
            pltpu.VMEM((TB, 1), jnp.float32),
        ],
        compiler_params=pltpu.CompilerParams(
            dimension_semantics=("parallel", "arbitrary"),
        ),
    )(x, mkf, G, W2, row(b1), row(g1), row(bt1), row(b2), row(g2), row(bt2),
      We1, row(be1), We2, row(be2))
    return mu, lv


# speedup vs baseline: 10.0801x; 1.1789x over previous
import jax
import jax.numpy as jnp
from jax.experimental import pallas as pl
from jax.experimental.pallas import tpu as pltpu

B, J, D, HH, EH, Z = 1024, 2048, 32, 128, 128, 64
TB, TJ = 8, 512
NB, NJ = B // TB, J // TJ


def _prep_kernel(F_ref, W10_ref, W1r_ref, b1_ref, g1_ref, Gp_ref, st_ref):
    F = F_ref[...]
    nrm = jnp.sqrt(jnp.sum(F * F, axis=1, keepdims=True))
    Fn = F / jnp.maximum(nrm, 1e-8)
    G = W10_ref[...] + jnp.dot(Fn, W1r_ref[...], preferred_element_type=jnp.float32)
    Gc = G - jnp.mean(G, axis=1, keepdims=True)
    Gp_ref[...] = Gc * g1_ref[...]
    b1 = b1_ref[...]
    b1c = b1 - jnp.mean(b1)
    vG = jnp.mean(Gc * Gc, axis=1, keepdims=True)
    cG = jnp.mean(Gc * b1c, axis=1, keepdims=True)
    st = jnp.concatenate([vG, cG], axis=1)
    st_ref[...] = st.T


def _ln_rows(v, eps=1e-5):
    m = jnp.mean(v, axis=1, keepdims=True)
    c = v - m
    var = jnp.mean(c * c, axis=1, keepdims=True)
    return c * jax.lax.rsqrt(var + eps)


def _main_kernel(x_ref, m_ref, Gp_ref, st_ref, W2_ref, b1_ref, g1_ref, bt1_ref,
                 b2_ref, g2_ref, bt2_ref, We1_ref, be1_ref, We2_ref, be2_ref,
                 mu_ref, lv_ref, acc, cnt):
    ij = pl.program_id(1)

    @pl.when(ij == 0)
    def _():
        acc[...] = jnp.zeros_like(acc)
        cnt[...] = jnp.zeros_like(cnt)

    xm = x_ref[...]
    mk = m_ref[...]
    gp = Gp_ref[...]
    vG = st_ref[0:1, :]
    cG = st_ref[1:2, :]
    b1 = b1_ref[...]
    b1c = b1 - jnp.mean(b1)
    vb1 = jnp.mean(b1c * b1c)
    b1p = (b1c * g1_ref[...]).reshape(1, 1, HH)
    bt1 = bt1_ref[...].reshape(1, 1, HH)

    var = xm * xm * vG + 2.0 * xm * cG + vb1
    r = jax.lax.rsqrt(var + 1e-5)
    v = xm[:, :, None] * gp[None, :, :]
    h1 = jnp.maximum((v + b1p) * r[:, :, None] + bt1, 0.0)
    h1 = h1.reshape(TB * TJ, HH).astype(jnp.bfloat16)
    h2 = jnp.dot(h1, W2_ref[...], preferred_element_type=jnp.float32)
    h2 = h2 + b2_ref[...]
    h2 = jnp.maximum(_ln_rows(h2) * g2_ref[...] + bt2_ref[...], 0.0)
    h2 = h2.reshape(TB, TJ, D) * mk[:, :, None]
    acc[...] += jnp.sum(h2, axis=1)
    cnt[...] += jnp.sum(mk, axis=1, keepdims=True)

    @pl.when(ij == NJ - 1)
    def _():
        pooled = acc[...] / jnp.maximum(cnt[...], 1.0)
        e = jnp.dot(pooled, We1_ref[...], preferred_element_type=jnp.float32)
        e = jnp.maximum(_ln_rows(e + be1_ref[...]), 0.0)
        e = jnp.dot(e, We2_ref[...], preferred_element_type=jnp.float32)
        e = jnp.maximum(_ln_rows(e + be2_ref[...]), 0.0)
        mu_ref[...] = e[:, :Z]
        lv_ref[...] = e[:, Z:]


@jax.jit
def kernel(x, mask, F_emb, W1, b1, g1, bt1, W2, b2, g2, bt2, We1, be1, We2, be2):
    row = lambda a: a.reshape(1, -1)
    Gp, st = pl.pallas_call(
        _prep_kernel,
        out_shape=[
            jax.ShapeDtypeStruct((J, HH), jnp.float32),
            jax.ShapeDtypeStruct((2, J), jnp.float32),
        ],
    )(F_emb, W1[0:1, :], W1[1:, :], row(b1), row(g1))

    mkf = mask.astype(jnp.float32)

    def const(shape):
        return pl.BlockSpec(shape, lambda ib, ij: (0, 0))

    mu, lv = pl.pallas_call(
        _main_kernel,
        grid=(NB, NJ),
        in_specs=[
            pl.BlockSpec((TB, TJ), lambda ib, ij: (ib, ij)),
            pl.BlockSpec((TB, TJ), lambda ib, ij: (ib, ij)),
            pl.BlockSpec((TJ, HH), lambda ib, ij: (ij, 0)),
            pl.BlockSpec((2, TJ), lambda ib, ij: (0, ij)),
            const((HH, D)),
            const((1, HH)), const((1, HH)), const((1, HH)),
            const((1, D)), const((1, D)), const((1, D)),
            const((D, EH)), const((1, EH)),
            const((EH, 2 * Z)), const((1, 2 * Z)),
        ],
        out_specs=[
            pl.BlockSpec((TB, Z), lambda ib, ij: (ib, 0)),
            pl.BlockSpec((TB, Z), lambda ib, ij: (ib, 0)),
        ],
        out_shape=[
            jax.ShapeDtypeStruct((B, Z), jnp.float32),
            jax.ShapeDtypeStruct((B, Z), jnp.float32),
        ],
        scratch_shapes=[
            pltpu.VMEM((TB, D), jnp.float32),
            pltpu.VMEM((TB, 1), jnp.float32),
        ],
        compiler_params=pltpu.CompilerParams(
            dimension_semantics=("parallel", "arbitrary"),
        ),
    )(x, mkf, Gp, st, W2.astype(jnp.bfloat16), row(b1), row(g1), row(bt1),
      row(b2), row(g2), row(bt2), We1, row(be1), We2, row(be2))
    return mu, lv
